# baseline (speedup 1.0000x reference)
import functools

import jax
import jax.numpy as jnp
from jax import lax
from jax.experimental import pallas as pl
from jax.experimental.pallas import tpu as pltpu
from jax.experimental.pallas import tpu_sc as plsc

B = 16384
F = 26
V = 100000
D = 32

NC = 2
NS = 16
NW = NC * NS

NPAIR = F * D
PER_W = NPAIR // NW
NCHUNK = 4
CB = B // NCHUNK

assert NPAIR % NW == 0
assert B % (NCHUNK * 16) == 0

_mesh = plsc.VectorSubcoreMesh(core_axis_name="c", subcore_axis_name="s")


@functools.partial(
    pl.kernel,
    mesh=_mesh,
    compiler_params=pltpu.CompilerParams(needs_layout_passes=False),
    out_type=jax.ShapeDtypeStruct((F, D, B), jnp.float32),
    scratch_types=[
        pltpu.VMEM((V,), jnp.float32),
        pltpu.VMEM((B,), jnp.int32),
        pltpu.VMEM((CB,), jnp.float32),
        pltpu.VMEM((CB,), jnp.float32),
        pltpu.SemaphoreType.DMA,
        pltpu.SemaphoreType.DMA,
    ],
)
def _lookup_kernel(xt_hbm, tt_hbm, out_hbm, tab_v, idx_v, out0_v, out1_v,
                   sem0, sem1):
    obuf = (out0_v, out1_v)
    osem = (sem0, sem1)
    nc = lax.axis_index("c")
    ns = lax.axis_index("s")
    wid = ns * NC + nc
    p0 = wid * PER_W

    def _pair(t, f_prev):
        p = p0 + t
        f = lax.div(p, D)
        d = lax.rem(p, D)

        @pl.when(f != f_prev)
        def _():
            pltpu.sync_copy(xt_hbm.at[f], idx_v)

        pltpu.sync_copy(tt_hbm.at[f, d], tab_v)

        for c in range(NCHUNK):
            bbuf = c % 2
            dst = out_hbm.at[f, d, pl.ds(c * CB, CB)]

            def _drain(dst=dst, bbuf=bbuf):
                pltpu.make_async_copy(obuf[bbuf], dst, osem[bbuf]).wait()

            if c < 2:
                pl.when(t > 0)(_drain)
            else:
                _drain()

            @plsc.parallel_loop(0, CB // 16, unroll=8)
            def _grp(j, c=c, bbuf=bbuf):
                idx = idx_v[pl.ds(c * CB + j * 16, 16)]
                obuf[bbuf][pl.ds(j * 16, 16)] = plsc.load_gather(tab_v, [idx])
            pltpu.async_copy(obuf[bbuf], dst, osem[bbuf])
        return f

    lax.fori_loop(0, PER_W, _pair, jnp.int32(-1))

    for bbuf in range(2):
        pltpu.make_async_copy(
            obuf[bbuf], out_hbm.at[0, 0, pl.ds(0, CB)], osem[bbuf]).wait()


def kernel(x, tables):
    xt = x.T
    tt = tables.transpose(0, 2, 1)
    ot = _lookup_kernel(xt, tt)
    return ot.transpose(2, 0, 1)
